# baseline (speedup 1.0000x reference)
import jax
import jax.numpy as jnp
from jax import lax
from jax.experimental import pallas as pl
from jax.experimental.pallas import tpu as pltpu
from jax.experimental.pallas import tpu_sc as plsc

_B = 8192
_T = 4
_N = 16
_E = 128
_ATT = 32

_NC, _NS = 2, 16
_NW = _NC * _NS
_CH = 128
_SPC = _CH // _N
_SEGS = _B * _T
_SEG_W = _SEGS // _NW
_NCH = _SEG_W // _SPC
_FLUSH = 8
_NE_W = _B // _NW
_NE_CH = _NE_W // _CH

_NSPLIT = 2
_B_C = _B // _NSPLIT
_SEG_C = _SEGS // _NSPLIT
_SEG_WC = _SEG_C // _NW
_NCH_C = _SEG_WC // _SPC
_NE_WC = _B_C // _NW
_NE_CH_C = _NE_WC // _CH


def _sc_gather_body(table, idx3, netab, neidx3, seg_out, ne_out,
                    idx_v, neidx_v, rows_v, outbuf, sem0, sem1):
    c = lax.axis_index("c")
    s = lax.axis_index("s")
    wid = s * _NC + c

    pltpu.sync_copy(idx3.at[wid], idx_v)
    pltpu.sync_copy(neidx3.at[wid], neidx_v)

    def idx_fix(j, carry):
        for g in range(_SPC):
            sl = pl.ds(g * _N, _N)
            idx_v[j, sl] = idx_v[j, sl] * _T + (g % _T)
        return carry

    lax.fori_loop(0, _NCH_C, idx_fix, 0)

    sems = (sem0, sem1)

    def fire(j, b):
        pltpu.async_copy(table.at[idx_v.at[j]], rows_v.at[b], sems[b])

    def wait(j, b):
        pltpu.make_async_copy(table.at[idx_v.at[j]], rows_v.at[b],
                              sems[b]).wait()

    def acc_chunk(j, b):
        def seg_body(sg, c2):
            r0 = sg * _N

            def n_body(n, acc):
                return tuple(acc[k] + rows_v[b, r0 + n, pl.ds(k * 16, 16)]
                             for k in range(_E // 16))

            acc0 = tuple(rows_v[b, r0, pl.ds(k * 16, 16)]
                         for k in range(_E // 16))
            acc = lax.fori_loop(1, _N, n_body, acc0)
            ob = lax.rem(j, _FLUSH) * _SPC + sg
            for k in range(_E // 16):
                outbuf[ob, pl.ds(k * 16, 16)] = acc[k]
            return c2

        lax.fori_loop(0, _SPC, seg_body, 0)

    fire(0, 0)

    def loop2(jj, carry):
        j0 = 2 * jj
        fire(j0 + 1, 1)
        wait(j0, 0)
        acc_chunk(j0, 0)

        @pl.when(jj + 1 < _NCH_C // 2)
        def _():
            fire(j0 + 2, 0)

        wait(j0 + 1, 1)
        acc_chunk(j0 + 1, 1)

        @pl.when(lax.rem(jj, _FLUSH // 2) == _FLUSH // 2 - 1)
        def _():
            seg_base = wid * _SEG_WC + (j0 + 1 - (_FLUSH - 1)) * _SPC
            pltpu.sync_copy(outbuf, seg_out.at[pl.ds(seg_base, _FLUSH * _SPC)])

        return carry

    lax.fori_loop(0, _NCH_C // 2, loop2, 0)

    for h in range(_NE_CH_C):
        b = h % 2
        pltpu.async_copy(netab.at[neidx_v.at[h]], rows_v.at[b], sems[b])
    for h in range(_NE_CH_C):
        b = h % 2
        pltpu.make_async_copy(netab.at[neidx_v.at[h]], rows_v.at[b],
                              sems[b]).wait()
        pltpu.sync_copy(rows_v.at[b],
                        ne_out.at[pl.ds(wid * _NE_WC + h * _CH, _CH)])


import functools


@functools.lru_cache(maxsize=1)
def _sc_gather_kernel():
    mesh = plsc.VectorSubcoreMesh(core_axis_name="c", subcore_axis_name="s",
                                  num_cores=_NC, num_subcores=_NS)
    return pl.kernel(
        _sc_gather_body,
        out_type=(jax.ShapeDtypeStruct((_SEG_C, _E), jnp.float32),
                  jax.ShapeDtypeStruct((_B_C, _E), jnp.float32)),
        mesh=mesh,
        scratch_types=[
            pltpu.VMEM((_NCH_C, _CH), jnp.int32),
            pltpu.VMEM((_NE_CH_C, _CH), jnp.int32),
            pltpu.VMEM((2, _CH, _E), jnp.float32),
            pltpu.VMEM((_FLUSH * _SPC, _E), jnp.float32),
            pltpu.SemaphoreType.DMA,
            pltpu.SemaphoreType.DMA,
        ],
    )

_BLK = 512


def _fexp(x):
    y = x * 1.4426950408889634
    k = jnp.floor(y + 0.5)
    c = x - k * 0.6931471805599453
    p = 1.0 + c * (1.0 + c * (0.5 + c * (0.16666667 + c * 0.041666668)))
    pi = jax.lax.bitcast_convert_type(p, jnp.int32)
    pi = pi + (k.astype(jnp.int32) << 23)
    return jax.lax.bitcast_convert_type(pi, jnp.float32)


def _tc_body(x_ref, ne_ref, oh_ref, s1c_ref, s2sel_ref, w_ref, o_ref):
    X = x_ref[...]
    Xf = X.reshape(_BLK * _T, _E)
    oh = oh_ref[...]

    M = jnp.tanh(jnp.dot(Xf, s1c_ref[...],
                         preferred_element_type=jnp.float32))
    Mw = M.reshape(_BLK, _T, _E) * s2sel_ref[...][:, None, :]

    ones = jnp.ones((_E, _E), jnp.float32)
    lb = jnp.dot(Mw.reshape(_BLK * _T, _E), ones,
                 preferred_element_type=jnp.float32)
    eb = _fexp(lb).reshape(_BLK, _T, _E)
    numer = jnp.sum(eb * X, axis=1)
    denom = jnp.sum(eb, axis=1)
    y = numer / denom

    delta = jnp.zeros((_BLK, _E), jnp.float32)
    for w in range(_T):
        delta = delta + oh[:, w:w + 1] * jnp.dot(
            y, w_ref[w], preferred_element_type=jnp.float32)

    out = ne_ref[...] + delta
    nrm2 = jnp.dot(out * out, ones, preferred_element_type=jnp.float32)
    o_ref[...] = out / jnp.maximum(jnp.sqrt(nrm2), 1e-12)


def _tc_combine(X, ne_rows, onehot, s1c, s2sel, w):
    grid = _B_C // _BLK
    return pl.pallas_call(
        _tc_body,
        grid=(grid,),
        in_specs=[
            pl.BlockSpec((_BLK, _T, _E), lambda i: (i, 0, 0)),
            pl.BlockSpec((_BLK, _E), lambda i: (i, 0)),
            pl.BlockSpec((_BLK, _T), lambda i: (i, 0)),
            pl.BlockSpec((_E, _E), lambda i: (0, 0)),
            pl.BlockSpec((_BLK, _E), lambda i: (i, 0)),
            pl.BlockSpec((_T, _E, _E), lambda i: (0, 0, 0)),
        ],
        out_specs=pl.BlockSpec((_BLK, _E), lambda i: (i, 0)),
        out_shape=jax.ShapeDtypeStruct((_B_C, _E), jnp.float32),
    )(X, ne_rows, onehot, s1c, s2sel, w)


def kernel(inputs, node_types, node_neigh, node_embeddings,
           node_type_embeddings, trans_weights, trans_weights_s1,
           trans_weights_s2):
    inputs = inputs.astype(jnp.int32)
    node_neigh = node_neigh.astype(jnp.int32)
    t_idx = jnp.arange(_T, dtype=jnp.int32)

    flat_idx = node_neigh.reshape(_NSPLIT, _NW, _NCH_C, _CH)
    ne_idx = inputs.reshape(_NSPLIT, _NW, _NE_CH_C, _CH)
    table_flat = node_type_embeddings.reshape(-1, _E)

    onehot = (node_types[:, None] == t_idx[None, :]).astype(jnp.float32)
    s1c = trans_weights_s1.transpose(1, 0, 2).reshape(_E, _T * _ATT)
    s2sel = (onehot[:, :, None]
             * trans_weights_s2[None, :, :, 0]).reshape(_B, _T * _ATT)

    sc = _sc_gather_kernel()
    outs = []
    for h in range(_NSPLIT):
        segsum, ne_rows = sc(table_flat, flat_idx[h], node_embeddings,
                             ne_idx[h])
        X = segsum.reshape(_B_C, _T, _E)
        lo = h * _B_C
        outs.append(_tc_combine(X, ne_rows, onehot[lo:lo + _B_C],
                                s1c, s2sel[lo:lo + _B_C], trans_weights))
    return jnp.concatenate(outs, axis=0)


# speedup vs baseline: 1.0393x; 1.0393x over previous
import jax
import jax.numpy as jnp
from jax import lax
from jax.experimental import pallas as pl
from jax.experimental.pallas import tpu as pltpu
from jax.experimental.pallas import tpu_sc as plsc

_B = 8192
_T = 4
_N = 16
_E = 128
_ATT = 32

_NC, _NS = 2, 16
_NW = _NC * _NS
_CH = 128
_SPC = _CH // _N
_SEGS = _B * _T
_SEG_W = _SEGS // _NW
_NCH = _SEG_W // _SPC
_FLUSH = 8
_NE_W = _B // _NW
_NE_CH = _NE_W // _CH

_NSPLIT = 2
_B_C = _B // _NSPLIT
_SEG_C = _SEGS // _NSPLIT
_SEG_WC = _SEG_C // _NW
_NCH_C = _SEG_WC // _SPC
_NE_WC = _B_C // _NW
_NE_CH_C = _NE_WC // _CH


def _sc_gather_body(table, idx3, netab, neidx3, seg_out, ne_out,
                    idx_v, neidx_v, rows_v, outbuf, sem0, sem1):
    c = lax.axis_index("c")
    s = lax.axis_index("s")
    wid = s * _NC + c

    pltpu.sync_copy(idx3.at[wid], idx_v)
    pltpu.sync_copy(neidx3.at[wid], neidx_v)

    sems = (sem0, sem1)

    def fire(j, b):
        pltpu.async_copy(table.at[idx_v.at[j]], rows_v.at[b], sems[b])

    def wait(j, b):
        pltpu.make_async_copy(table.at[idx_v.at[j]], rows_v.at[b],
                              sems[b]).wait()

    def acc_chunk(j, b):
        def seg_body(sg, c2):
            r0 = sg * _N

            def n_body(n, acc):
                return tuple(acc[k] + rows_v[b, r0 + n, pl.ds(k * 16, 16)]
                             for k in range(_E // 16))

            acc0 = tuple(rows_v[b, r0, pl.ds(k * 16, 16)]
                         for k in range(_E // 16))
            acc = lax.fori_loop(1, _N, n_body, acc0)
            ob = lax.rem(j, _FLUSH) * _SPC + sg
            for k in range(_E // 16):
                outbuf[ob, pl.ds(k * 16, 16)] = acc[k]
            return c2

        lax.fori_loop(0, _SPC, seg_body, 0)

    fire(0, 0)

    def loop2(jj, carry):
        j0 = 2 * jj
        fire(j0 + 1, 1)
        wait(j0, 0)
        acc_chunk(j0, 0)

        @pl.when(jj + 1 < _NCH_C // 2)
        def _():
            fire(j0 + 2, 0)

        wait(j0 + 1, 1)
        acc_chunk(j0 + 1, 1)

        @pl.when(lax.rem(jj, _FLUSH // 2) == _FLUSH // 2 - 1)
        def _():
            seg_base = wid * _SEG_WC + (j0 + 1 - (_FLUSH - 1)) * _SPC
            pltpu.sync_copy(outbuf, seg_out.at[pl.ds(seg_base, _FLUSH * _SPC)])

        return carry

    lax.fori_loop(0, _NCH_C // 2, loop2, 0)

    for h in range(_NE_CH_C):
        b = h % 2
        pltpu.async_copy(netab.at[neidx_v.at[h]], rows_v.at[b], sems[b])
    for h in range(_NE_CH_C):
        b = h % 2
        pltpu.make_async_copy(netab.at[neidx_v.at[h]], rows_v.at[b],
                              sems[b]).wait()
        pltpu.sync_copy(rows_v.at[b],
                        ne_out.at[pl.ds(wid * _NE_WC + h * _CH, _CH)])


import functools


@functools.lru_cache(maxsize=1)
def _sc_gather_kernel():
    mesh = plsc.VectorSubcoreMesh(core_axis_name="c", subcore_axis_name="s",
                                  num_cores=_NC, num_subcores=_NS)
    return pl.kernel(
        _sc_gather_body,
        out_type=(jax.ShapeDtypeStruct((_SEG_C, _E), jnp.float32),
                  jax.ShapeDtypeStruct((_B_C, _E), jnp.float32)),
        mesh=mesh,
        scratch_types=[
            pltpu.VMEM((_NCH_C, _CH), jnp.int32),
            pltpu.VMEM((_NE_CH_C, _CH), jnp.int32),
            pltpu.VMEM((2, _CH, _E), jnp.float32),
            pltpu.VMEM((_FLUSH * _SPC, _E), jnp.float32),
            pltpu.SemaphoreType.DMA,
            pltpu.SemaphoreType.DMA,
        ],
    )

_BLK = 512


def _tc_body(x_ref, ne_ref, oh_ref, s1c_ref, s2sel_ref, w_ref, o_ref):
    X = x_ref[...]
    Xf = X.reshape(_BLK * _T, _E)
    oh = oh_ref[...]

    M = jnp.tanh(jnp.dot(Xf, s1c_ref[...],
                         preferred_element_type=jnp.float32))
    Mw = M.reshape(_BLK, _T, _E) * s2sel_ref[...][:, None, :]

    ones = jnp.ones((_E, _E), jnp.float32)
    lb = jnp.dot(Mw.reshape(_BLK * _T, _E), ones,
                 preferred_element_type=jnp.float32)
    eb = jnp.exp(lb).reshape(_BLK, _T, _E)
    numer = jnp.sum(eb * X, axis=1)
    denom = jnp.sum(eb, axis=1)
    y = numer / denom

    delta = jnp.zeros((_BLK, _E), jnp.float32)
    for w in range(_T):
        delta = delta + oh[:, w:w + 1] * jnp.dot(
            y, w_ref[w], preferred_element_type=jnp.float32)

    out = ne_ref[...] + delta
    nrm2 = jnp.dot(out * out, ones, preferred_element_type=jnp.float32)
    o_ref[...] = out / jnp.maximum(jnp.sqrt(nrm2), 1e-12)


def _tc_combine(X, ne_rows, onehot, s1c, s2sel, w):
    grid = _B_C // _BLK
    return pl.pallas_call(
        _tc_body,
        grid=(grid,),
        in_specs=[
            pl.BlockSpec((_BLK, _T, _E), lambda i: (i, 0, 0)),
            pl.BlockSpec((_BLK, _E), lambda i: (i, 0)),
            pl.BlockSpec((_BLK, _T), lambda i: (i, 0)),
            pl.BlockSpec((_E, _E), lambda i: (0, 0)),
            pl.BlockSpec((_BLK, _E), lambda i: (i, 0)),
            pl.BlockSpec((_T, _E, _E), lambda i: (0, 0, 0)),
        ],
        out_specs=pl.BlockSpec((_BLK, _E), lambda i: (i, 0)),
        out_shape=jax.ShapeDtypeStruct((_B_C, _E), jnp.float32),
    )(X, ne_rows, onehot, s1c, s2sel, w)


def kernel(inputs, node_types, node_neigh, node_embeddings,
           node_type_embeddings, trans_weights, trans_weights_s1,
           trans_weights_s2):
    inputs = inputs.astype(jnp.int32)
    node_neigh = node_neigh.astype(jnp.int32)
    t_idx = jnp.arange(_T, dtype=jnp.int32)

    flat_idx = (node_neigh * _T
                + t_idx[None, :, None]).reshape(_NSPLIT, _NW, _NCH_C, _CH)
    ne_idx = inputs.reshape(_NSPLIT, _NW, _NE_CH_C, _CH)
    table_flat = node_type_embeddings.reshape(-1, _E)

    onehot = (node_types[:, None] == t_idx[None, :]).astype(jnp.float32)
    s1c = trans_weights_s1.transpose(1, 0, 2).reshape(_E, _T * _ATT)
    s2sel = (onehot[:, :, None]
             * trans_weights_s2[None, :, :, 0]).reshape(_B, _T * _ATT)

    sc = _sc_gather_kernel()
    outs = []
    for h in range(_NSPLIT):
        segsum, ne_rows = sc(table_flat, flat_idx[h], node_embeddings,
                             ne_idx[h])
        X = segsum.reshape(_B_C, _T, _E)
        lo = h * _B_C
        outs.append(_tc_combine(X, ne_rows, onehot[lo:lo + _B_C],
                                s1c, s2sel[lo:lo + _B_C], trans_weights))
    return jnp.concatenate(outs, axis=0)
